# Initial kernel scaffold; baseline (speedup 1.0000x reference)
import jax
import jax.numpy as jnp
from jax.experimental import pallas as pl


def kernel(pos, positional_embeddings):
    raise NotImplementedError("write your pallas kernel here")


# speedup vs baseline: 1.4473x; 1.4473x over previous
import functools

import jax
import jax.numpy as jnp
import numpy as np
from jax import lax
from jax.experimental import pallas as pl
from jax.experimental.pallas import tpu as pltpu
from jax.experimental.pallas import tpu_sc as plsc

HIDDEN = 768
NUM_POS = 577
WIDTH = 24
N = 131072

NC = 2
NS = 16
NW = NC * NS
RPW = N // NW
C = 64
NCH = RPW // C
NVEC = RPW // 16

_DENOM = np.float32(2.0 + 1e-6)


def _sc_body(pos_hbm, table_hbm, out_hbm, pos_v, idx_v, rows_v, sem):
    wid = lax.axis_index("s") * NC + lax.axis_index("c")
    base = wid * RPW

    pltpu.sync_copy(pos_hbm.at[pl.ds(base * 2, 2 * RPW)], pos_v)

    lane = lax.iota(jnp.int32, 16)
    even = lane * 2

    @pl.loop(0, NCH)
    def _compute(ch):
        for s in range(C // 16):
            off = (ch * (C // 16) + s) * 32
            xs = plsc.load_gather(pos_v, [off + even])
            ys = plsc.load_gather(pos_v, [off + even + 1])
            fx = (((xs + 1.0) / _DENOM) * np.float32(WIDTH)).astype(jnp.int32)
            fy = (((ys + 1.0) / _DENOM) * np.float32(WIDTH)).astype(jnp.int32)
            idx_v[ch, pl.ds(s * 16, 16)] = fx * WIDTH + fy

    @pl.loop(0, NCH)
    def _move(ch):
        pltpu.async_copy(table_hbm.at[idx_v.at[ch]], rows_v, sem).wait()
        pltpu.sync_copy(rows_v, out_hbm.at[pl.ds(base + ch * C, C)])


@jax.jit
def _sc_embed(pos_flat, table):
    mesh = plsc.VectorSubcoreMesh(
        core_axis_name="c", subcore_axis_name="s", num_cores=NC, num_subcores=NS
    )
    return pl.kernel(
        _sc_body,
        out_type=jax.ShapeDtypeStruct((N, HIDDEN), jnp.float32),
        mesh=mesh,
        scratch_types=[
            pltpu.VMEM((2 * RPW,), jnp.float32),
            pltpu.VMEM((NCH, C), jnp.int32),
            pltpu.VMEM((C, HIDDEN), jnp.float32),
            pltpu.SemaphoreType.DMA,
        ],
        compiler_params=pltpu.CompilerParams(needs_layout_passes=False),
    )(pos_flat, table)


def kernel(pos, positional_embeddings):
    pos_flat = pos.reshape(N * 2)
    table = positional_embeddings.reshape(NUM_POS, HIDDEN)
    out = _sc_embed(pos_flat, table)
    return out.reshape(1, N, HIDDEN)
